# Initial kernel scaffold; baseline (speedup 1.0000x reference)
import jax
import jax.numpy as jnp
from jax.experimental import pallas as pl


def kernel(items, table):
    raise NotImplementedError("write your pallas kernel here")


# speedup vs baseline: 1.4980x; 1.4980x over previous
import functools

import jax
import jax.numpy as jnp
from jax import lax
from jax.experimental import pallas as pl
from jax.experimental.pallas import tpu as pltpu
from jax.experimental.pallas import tpu_sc as plsc

VOCAB = 1000000
EMBED_DIM = 32
NUM_IDX = 4096 * 200
NC, NS = 2, 16
NW = NC * NS
PER_W = NUM_IDX // NW
CHUNK = 1600
NCH = PER_W // CHUNK

_mesh = plsc.VectorSubcoreMesh(
    core_axis_name="c", subcore_axis_name="s", num_cores=NC, num_subcores=NS
)


def _body(idx_hbm, table_hbm, out_hbm, idx_v0, idx_v1, rows_v0, rows_v1,
          gsem0, gsem1, osem0, osem1):
    wid = lax.axis_index("s") * NC + lax.axis_index("c")
    base = wid * PER_W
    idx_v = (idx_v0, idx_v1)
    rows_v = (rows_v0, rows_v1)
    gsems = (gsem0, gsem1)
    osems = (osem0, osem1)

    def load_idx(j, b):
        pltpu.sync_copy(idx_hbm.at[pl.ds(base + j * CHUNK, CHUNK)], idx_v[b])

    def start_gather(b):
        return pltpu.async_copy(table_hbm.at[idx_v[b]], rows_v[b], gsems[b])

    def start_store(j, b):
        return pltpu.async_copy(
            rows_v[b], out_hbm.at[pl.ds(base + j * CHUNK, CHUNK)], osems[b]
        )

    gathers = [None, None]
    stores = [None, None]
    load_idx(0, 0)
    gathers[0] = start_gather(0)
    for j in range(NCH):
        b = j & 1
        nb = 1 - b
        if j + 1 < NCH:
            load_idx(j + 1, nb)
            if stores[nb] is not None:
                stores[nb].wait()
            gathers[nb] = start_gather(nb)
        gathers[b].wait()
        stores[b] = start_store(j, b)
    stores[0].wait()
    stores[1].wait()


_gather_call = pl.kernel(
    _body,
    out_type=jax.ShapeDtypeStruct((NUM_IDX, EMBED_DIM), jnp.float32),
    mesh=_mesh,
    scratch_types=[
        pltpu.VMEM((CHUNK,), jnp.int32),
        pltpu.VMEM((CHUNK,), jnp.int32),
        pltpu.VMEM((CHUNK, EMBED_DIM), jnp.float32),
        pltpu.VMEM((CHUNK, EMBED_DIM), jnp.float32),
        pltpu.SemaphoreType.DMA,
        pltpu.SemaphoreType.DMA,
        pltpu.SemaphoreType.DMA,
        pltpu.SemaphoreType.DMA,
    ],
    compiler_params=pltpu.CompilerParams(use_tc_tiling_on_sc=False),
)


@jax.jit
def kernel(items, table):
    idx = items.reshape(-1).astype(jnp.int32)
    out = _gather_call(idx, table)
    return out.reshape(items.shape + (table.shape[1],))
